# baseline (speedup 1.0000x reference)
import functools

import jax
import jax.numpy as jnp
from jax import lax
from jax.experimental import pallas as pl
from jax.experimental.pallas import tpu as pltpu
from jax.experimental.pallas import tpu_sc as plsc

_NUM_WORKERS = 32
_LANES = 128


def _make_emb_kernel(bsz, seq, d, v2):
    mesh = plsc.VectorSubcoreMesh(core_axis_name="c", subcore_axis_name="s")
    rpp = _LANES // d

    @functools.partial(
        pl.kernel,
        mesh=mesh,
        out_type=jax.ShapeDtypeStruct((seq, d, bsz), jnp.float32),
        scratch_types=[
            pltpu.VMEM((seq, _LANES), jnp.int32),
            pltpu.VMEM((seq, _LANES), jnp.int32),
            pltpu.VMEM((3, _LANES, _LANES), jnp.float32),
            pltpu.VMEM((2, d, _LANES), jnp.float32),
            pltpu.SemaphoreType.DMA((3,)),
            pltpu.SemaphoreType.DMA((2,)),
        ],
        compiler_params=pltpu.CompilerParams(needs_layout_passes=False),
    )
    def emb(xg_hbm, xq_hbm, tab_hbm, out_hbm, idxg_v, idxq_v, rows_v, tbuf_v,
            gsem, osem):
        wid = lax.axis_index("s") * 2 + lax.axis_index("c")
        lane0 = wid * _LANES
        pltpu.sync_copy(xg_hbm.at[:, pl.ds(lane0, _LANES)], idxg_v)
        pltpu.sync_copy(xq_hbm.at[:, pl.ds(lane0, _LANES)], idxq_v)

        iot = lax.iota(jnp.int32, 16)

        def fire_gather(s, rb):
            pltpu.async_copy(tab_hbm.at[idxg_v.at[s]], rows_v.at[rb],
                             gsem.at[rb])

        def wait_gather(rb):
            pltpu.make_async_copy(
                tab_hbm.at[idxg_v.at[0]], rows_v.at[rb], gsem.at[rb]).wait()

        def fire_out(s, tb):
            pltpu.async_copy(
                tbuf_v.at[tb], out_hbm.at[s, :, pl.ds(lane0, _LANES)],
                osem.at[tb])

        def wait_out(tb):
            pltpu.make_async_copy(
                tbuf_v.at[tb], out_hbm.at[0, :, pl.ds(lane0, _LANES)],
                osem.at[tb]).wait()

        def transpose_chunk(s, rb, tb):
            cols0 = []
            for blk in range(_LANES // 16):
                q = idxq_v[s, pl.ds(blk * 16, 16)]
                cols0.append((q & (rpp - 1)) * d)

            def dbody(dd, carry):
                for blk in range(_LANES // 16):
                    val = plsc.load_gather(
                        rows_v.at[rb], [iot + blk * 16, cols0[blk] + dd])
                    tbuf_v[tb, dd, pl.ds(blk * 16, 16)] = val
                return carry

            lax.fori_loop(0, d, dbody, 0)

        def step(s, rb, tb, fire_ahead=True, wait_o=True):
            if wait_o:
                wait_out(tb)
            if fire_ahead:
                fire_gather(s + 2, (rb + 2) % 3)
            wait_gather(rb)
            transpose_chunk(s, rb, tb)
            fire_out(s, tb)

        fire_gather(0, 0)
        fire_gather(1, 1)
        step(0, 0, 0, wait_o=False)
        step(1, 1, 1, wait_o=False)

        n_groups = (seq - 8) // 6
        assert (seq - 8) % 6 == 0

        def group(k, _):
            g = 2 + 6 * k
            for u in range(6):
                step(g + u, (2 + u) % 3, u % 2)
            return 0

        lax.fori_loop(0, n_groups, group, 0)

        for s in range(seq - 6, seq - 2):
            step(s, s % 3, s % 2)
        for s in range(seq - 2, seq):
            step(s, s % 3, s % 2, fire_ahead=False)
        for tb in range(2):
            wait_out(tb)

    return emb


def kernel(x, table):
    bsz, seq = x.shape
    v, d = table.shape
    rpp = _LANES // d
    assert bsz == _NUM_WORKERS * _LANES and v % rpp == 0 and d * rpp == _LANES
    tab2 = table.reshape(v // rpp, _LANES)
    xt = x.T.astype(jnp.int32)
    xg = xt // rpp
    out3 = _make_emb_kernel(bsz, seq, d, v // rpp)(xg, xt, tab2)
    return out3.transpose(2, 0, 1)


# speedup vs baseline: 1.0045x; 1.0045x over previous
import functools

import jax
import jax.numpy as jnp
from jax import lax
from jax.experimental import pallas as pl
from jax.experimental.pallas import tpu as pltpu
from jax.experimental.pallas import tpu_sc as plsc

_NUM_WORKERS = 32
_LANES = 128


def _make_emb_kernel(bsz, seq, d):
    mesh = plsc.VectorSubcoreMesh(core_axis_name="c", subcore_axis_name="s")
    ndg = d // 8
    nblk = _LANES // 16

    @functools.partial(
        pl.kernel,
        mesh=mesh,
        out_type=jax.ShapeDtypeStruct((seq, ndg, _NUM_WORKERS, 8, _LANES),
                                      jnp.float32),
        scratch_types=[
            pltpu.VMEM((seq, _LANES), jnp.int32),
            pltpu.VMEM((3, _LANES, d), jnp.float32),
            pltpu.VMEM((2, d, _LANES), jnp.float32),
            pltpu.SemaphoreType.DMA((3,)),
            pltpu.SemaphoreType.DMA((2,)),
        ],
        compiler_params=pltpu.CompilerParams(
            use_tc_tiling_on_sc=False, needs_layout_passes=False),
    )
    def emb(x_hbm, tab_hbm, out_hbm, idx_v, rows_v, tbuf_v, gsem, osem):
        wid = lax.axis_index("s") * 2 + lax.axis_index("c")
        lane0 = wid * _LANES
        pltpu.sync_copy(x_hbm.at[:, pl.ds(lane0, _LANES)], idx_v)

        iot = lax.iota(jnp.int32, 16)

        def fire_gather(s, rb):
            pltpu.async_copy(tab_hbm.at[idx_v.at[s]], rows_v.at[rb],
                             gsem.at[rb])

        def wait_gather(rb):
            pltpu.make_async_copy(
                tab_hbm.at[idx_v.at[0]], rows_v.at[rb], gsem.at[rb]).wait()

        def fire_out(s, tb):
            for dg in range(ndg):
                pltpu.async_copy(
                    tbuf_v.at[tb, pl.ds(dg * 8, 8)],
                    out_hbm.at[s, dg, wid], osem.at[tb])

        def wait_out(tb):
            for dg in range(ndg):
                pltpu.make_async_copy(
                    tbuf_v.at[tb, pl.ds(dg * 8, 8)],
                    out_hbm.at[0, dg, wid], osem.at[tb]).wait()

        def transpose_chunk(rb, tb):
            def dbody(dd, carry):
                col = lax.broadcast(dd, (16,))
                for blk in range(nblk):
                    val = plsc.load_gather(
                        rows_v.at[rb], [iot + blk * 16, col])
                    tbuf_v[tb, dd, pl.ds(blk * 16, 16)] = val
                return carry

            lax.fori_loop(0, d, dbody, 0)

        def step(s, rb, tb, fire_ahead=True, wait_o=True):
            if wait_o:
                wait_out(tb)
            if fire_ahead:
                fire_gather(s + 2, (rb + 2) % 3)
            wait_gather(rb)
            transpose_chunk(rb, tb)
            fire_out(s, tb)

        fire_gather(0, 0)
        fire_gather(1, 1)
        step(0, 0, 0, wait_o=False)
        step(1, 1, 1, wait_o=False)

        n_groups = (seq - 8) // 6
        assert (seq - 8) % 6 == 0

        def group(k, _):
            g = 2 + 6 * k
            for u in range(6):
                step(g + u, (2 + u) % 3, u % 2)
            return 0

        lax.fori_loop(0, n_groups, group, 0)

        for s in range(seq - 6, seq - 2):
            step(s, s % 3, s % 2)
        for s in range(seq - 2, seq):
            step(s, s % 3, s % 2, fire_ahead=False)
        for tb in range(2):
            wait_out(tb)

    return emb


def kernel(x, table):
    bsz, seq = x.shape
    v, d = table.shape
    assert bsz == _NUM_WORKERS * _LANES and d % 8 == 0
    xt = x.T.astype(jnp.int32)
    out5 = _make_emb_kernel(bsz, seq, d)(xt, table)
    out3 = out5.transpose(0, 1, 3, 2, 4).reshape(seq, d, bsz)
    return out3.transpose(2, 0, 1)


# speedup vs baseline: 1.1263x; 1.1213x over previous
import functools

import jax
import jax.numpy as jnp
from jax import lax
from jax.experimental import pallas as pl
from jax.experimental.pallas import tpu as pltpu
from jax.experimental.pallas import tpu_sc as plsc

_NUM_WORKERS = 32
_LANES = 128


def _make_emb_kernel(bsz, seq, d):
    mesh = plsc.VectorSubcoreMesh(core_axis_name="c", subcore_axis_name="s")
    ndg = d // 8
    nblk = _LANES // 16

    @functools.partial(
        pl.kernel,
        mesh=mesh,
        out_type=jax.ShapeDtypeStruct((seq, ndg, _NUM_WORKERS, 8, _LANES),
                                      jnp.float32),
        scratch_types=[
            pltpu.VMEM((seq, _LANES), jnp.int32),
            pltpu.VMEM((3, _LANES, d), jnp.float32),
            pltpu.VMEM((2, d, _LANES), jnp.float32),
            pltpu.SemaphoreType.DMA((3,)),
            pltpu.SemaphoreType.DMA((2,)),
        ],
        compiler_params=pltpu.CompilerParams(
            use_tc_tiling_on_sc=False, needs_layout_passes=False),
    )
    def emb(x_hbm, tab_hbm, out_hbm, idx_v, rows_v, tbuf_v, gsem, osem):
        wid = lax.axis_index("s") * 2 + lax.axis_index("c")
        lane0 = wid * _LANES
        pltpu.sync_copy(x_hbm.at[:, pl.ds(lane0, _LANES)], idx_v)

        iot = lax.iota(jnp.int32, 16)
        iotbs = [iot + blk * 16 for blk in range(nblk)]

        def fire_gather(s, rb):
            pltpu.async_copy(tab_hbm.at[idx_v.at[s]], rows_v.at[rb],
                             gsem.at[rb])

        def wait_gather(rb):
            pltpu.make_async_copy(
                tab_hbm.at[idx_v.at[0]], rows_v.at[rb], gsem.at[rb]).wait()

        def fire_out(s, tb):
            for dg in range(ndg):
                pltpu.async_copy(
                    tbuf_v.at[tb, pl.ds(dg * 8, 8)],
                    out_hbm.at[s, dg, wid], osem.at[tb])

        def wait_out(tb):
            for dg in range(ndg):
                pltpu.make_async_copy(
                    tbuf_v.at[tb, pl.ds(dg * 8, 8)],
                    out_hbm.at[0, dg, wid], osem.at[tb]).wait()

        def transpose_chunk(rb, tb):
            def dbody(i, carry):
                dd = i * 2
                vals = []
                for du in range(2):
                    col = lax.broadcast(dd + du, (16,))
                    for blk in range(nblk):
                        vals.append(plsc.load_gather(
                            rows_v.at[rb], [iotbs[blk], col]))
                for du in range(2):
                    for blk in range(nblk):
                        tbuf_v[tb, dd + du, pl.ds(blk * 16, 16)] = (
                            vals[du * nblk + blk])
                return carry

            lax.fori_loop(0, d // 2, dbody, 0)

        def step(s, rb, tb, fire_ahead=True, wait_o=True):
            if wait_o:
                wait_out(tb)
            if fire_ahead:
                fire_gather(s + 2, (rb + 2) % 3)
            wait_gather(rb)
            transpose_chunk(rb, tb)
            fire_out(s, tb)

        fire_gather(0, 0)
        fire_gather(1, 1)
        step(0, 0, 0, wait_o=False)
        step(1, 1, 1, wait_o=False)

        n_groups = (seq - 8) // 6
        assert (seq - 8) % 6 == 0

        def group(k, _):
            g = 2 + 6 * k
            for u in range(6):
                step(g + u, (2 + u) % 3, u % 2)
            return 0

        lax.fori_loop(0, n_groups, group, 0)

        for s in range(seq - 6, seq - 2):
            step(s, s % 3, s % 2)
        for s in range(seq - 2, seq):
            step(s, s % 3, s % 2, fire_ahead=False)
        for tb in range(2):
            wait_out(tb)

    return emb


def kernel(x, table):
    bsz, seq = x.shape
    v, d = table.shape
    assert bsz == _NUM_WORKERS * _LANES and d % 8 == 0
    xt = x.T.astype(jnp.int32)
    out5 = _make_emb_kernel(bsz, seq, d)(xt, table)
    out3 = out5.transpose(0, 1, 3, 2, 4).reshape(seq, d, bsz)
    return out3.transpose(2, 0, 1)


# speedup vs baseline: 1.1297x; 1.0030x over previous
import functools

import jax
import jax.numpy as jnp
from jax import lax
from jax.experimental import pallas as pl
from jax.experimental.pallas import tpu as pltpu
from jax.experimental.pallas import tpu_sc as plsc

_NUM_WORKERS = 32
_LANES = 128


def _make_emb_kernel(bsz, seq, d):
    mesh = plsc.VectorSubcoreMesh(core_axis_name="c", subcore_axis_name="s")
    ndg = d // 8
    nblk = _LANES // 16

    @functools.partial(
        pl.kernel,
        mesh=mesh,
        out_type=jax.ShapeDtypeStruct((seq, ndg, _NUM_WORKERS, 8, _LANES),
                                      jnp.float32),
        scratch_types=[
            pltpu.VMEM((seq, _LANES), jnp.int32),
            pltpu.VMEM((3, _LANES, d), jnp.float32),
            pltpu.VMEM((2, d, _LANES), jnp.float32),
            pltpu.SemaphoreType.DMA((3,)),
            pltpu.SemaphoreType.DMA((2,)),
        ],
        compiler_params=pltpu.CompilerParams(
            use_tc_tiling_on_sc=False, needs_layout_passes=False),
    )
    def emb(x_hbm, tab_hbm, out_hbm, idx_v, rows_v, tbuf_v, gsem, osem):
        wid = lax.axis_index("s") * 2 + lax.axis_index("c")
        lane0 = wid * _LANES
        pltpu.sync_copy(x_hbm.at[:, pl.ds(lane0, _LANES)], idx_v)

        iot = lax.iota(jnp.int32, 16)
        iotbs = [iot + blk * 16 for blk in range(nblk)]

        def fire_gather(s, rb):
            pltpu.async_copy(tab_hbm.at[idx_v.at[s]], rows_v.at[rb],
                             gsem.at[rb])

        def wait_gather(rb):
            pltpu.make_async_copy(
                tab_hbm.at[idx_v.at[0]], rows_v.at[rb], gsem.at[rb]).wait()

        def fire_out(s, tb):
            for dg in range(ndg):
                pltpu.async_copy(
                    tbuf_v.at[tb, pl.ds(dg * 8, 8)],
                    out_hbm.at[s, dg, wid], osem.at[tb])

        def wait_out(tb):
            for dg in range(ndg):
                pltpu.make_async_copy(
                    tbuf_v.at[tb, pl.ds(dg * 8, 8)],
                    out_hbm.at[0, dg, wid], osem.at[tb]).wait()

        def transpose_chunk(rb, tb):
            def dbody(i, carry):
                dd = i * 4
                vals = []
                for du in range(4):
                    col = lax.broadcast(dd + du, (16,))
                    for blk in range(nblk):
                        vals.append(plsc.load_gather(
                            rows_v.at[rb], [iotbs[blk], col]))
                for du in range(4):
                    for blk in range(nblk):
                        tbuf_v[tb, dd + du, pl.ds(blk * 16, 16)] = (
                            vals[du * nblk + blk])
                return carry

            lax.fori_loop(0, d // 4, dbody, 0)

        def step(s, rb, tb, fire_ahead=True, wait_o=True):
            if wait_o:
                wait_out(tb)
            if fire_ahead:
                fire_gather(s + 2, (rb + 2) % 3)
            wait_gather(rb)
            transpose_chunk(rb, tb)
            fire_out(s, tb)

        fire_gather(0, 0)
        fire_gather(1, 1)
        step(0, 0, 0, wait_o=False)
        step(1, 1, 1, wait_o=False)

        n_groups = (seq - 8) // 6
        assert (seq - 8) % 6 == 0

        def group(k, _):
            g = 2 + 6 * k
            for u in range(6):
                step(g + u, (2 + u) % 3, u % 2)
            return 0

        lax.fori_loop(0, n_groups, group, 0)

        for s in range(seq - 6, seq - 2):
            step(s, s % 3, s % 2)
        for s in range(seq - 2, seq):
            step(s, s % 3, s % 2, fire_ahead=False)
        for tb in range(2):
            wait_out(tb)

    return emb


def kernel(x, table):
    bsz, seq = x.shape
    v, d = table.shape
    assert bsz == _NUM_WORKERS * _LANES and d % 8 == 0
    xt = x.T.astype(jnp.int32)
    out5 = _make_emb_kernel(bsz, seq, d)(xt, table)
    out3 = out5.transpose(0, 1, 3, 2, 4).reshape(seq, d, bsz)
    return out3.transpose(2, 0, 1)
